# Initial kernel scaffold; baseline (speedup 1.0000x reference)
import jax
import jax.numpy as jnp
from jax.experimental import pallas as pl


def kernel(x, table):
    raise NotImplementedError("write your pallas kernel here")


# speedup vs baseline: 1.2092x; 1.2092x over previous
import functools

import jax
import jax.numpy as jnp
from jax import lax
from jax.experimental import pallas as pl
from jax.experimental.pallas import tpu as pltpu
from jax.experimental.pallas import tpu_sc as plsc

_LANE = 128


@functools.lru_cache(maxsize=None)
def _make_gather(n_rows, d):
    info = plsc.get_sparse_core_info()
    num_cores, num_subcores = info.num_cores, info.num_subcores
    nw = num_cores * num_subcores
    rows_per_w = n_rows // nw
    chunks = rows_per_w // _LANE
    assert chunks * _LANE * nw == n_rows

    mesh = plsc.VectorSubcoreMesh(core_axis_name="c", subcore_axis_name="s")

    @functools.partial(
        pl.kernel,
        mesh=mesh,
        out_type=jax.ShapeDtypeStruct((n_rows, d), jnp.float32),
        scratch_types=[
            pltpu.VMEM((chunks, _LANE), jnp.int32),
            pltpu.VMEM((_LANE, d), jnp.float32),
            pltpu.SemaphoreType.DMA,
        ],
    )
    def k(table_hbm, idx_hbm, out_hbm, idx_v, rows_v, sem):
        wid = lax.axis_index("s") * num_cores + lax.axis_index("c")
        base_row = wid * chunks
        pltpu.sync_copy(idx_hbm.at[wid], idx_v)

        def body(c, carry):
            pltpu.async_copy(table_hbm.at[idx_v.at[c]], rows_v, sem).wait()
            pltpu.sync_copy(rows_v, out_hbm.at[pl.ds((base_row + c) * _LANE, _LANE)])
            return carry

        lax.fori_loop(0, chunks, body, 0)

    return k


def kernel(x, table):
    b, s = x.shape
    n = b * s
    d = table.shape[1]
    nw = 32
    idx = x.reshape(nw, n // (nw * _LANE), _LANE).astype(jnp.int32)
    out = _make_gather(n, d)(table, idx)
    return out.reshape(b, s, d)


# speedup vs baseline: 1.2778x; 1.0567x over previous
import functools

import jax
import jax.numpy as jnp
from jax import lax
from jax.experimental import pallas as pl
from jax.experimental.pallas import tpu as pltpu
from jax.experimental.pallas import tpu_sc as plsc

_LANE = 128


@functools.lru_cache(maxsize=None)
def _make_gather(n_rows, d):
    info = plsc.get_sparse_core_info()
    num_cores, num_subcores = info.num_cores, info.num_subcores
    nw = num_cores * num_subcores
    rows_per_w = n_rows // nw
    chunks = rows_per_w // _LANE
    assert chunks * _LANE * nw == n_rows
    assert chunks % 2 == 0 and chunks >= 4

    mesh = plsc.VectorSubcoreMesh(core_axis_name="c", subcore_axis_name="s")

    @functools.partial(
        pl.kernel,
        mesh=mesh,
        out_type=jax.ShapeDtypeStruct((n_rows, d), jnp.float32),
        scratch_types=[
            pltpu.VMEM((chunks, _LANE), jnp.int32),
            pltpu.VMEM((2, _LANE, d), jnp.float32),
            pltpu.SemaphoreType.DMA,
            pltpu.SemaphoreType.DMA,
            pltpu.SemaphoreType.DMA,
            pltpu.SemaphoreType.DMA,
        ],
    )
    def k(table_hbm, idx_hbm, out_hbm, idx_v, rows_v, g0, g1, o0, o1):
        wid = lax.axis_index("s") * num_cores + lax.axis_index("c")
        base_row = wid * chunks
        pltpu.sync_copy(idx_hbm.at[wid], idx_v)

        gsem = (g0, g1)
        osem = (o0, o1)

        def gather(c, b, sem):
            return pltpu.make_async_copy(
                table_hbm.at[idx_v.at[c]], rows_v.at[b], sem)

        def out_copy(c, b, sem):
            return pltpu.make_async_copy(
                rows_v.at[b], out_hbm.at[pl.ds((base_row + c) * _LANE, _LANE)], sem)

        gather(0, 0, g0).start()
        gather(0, 0, g0).wait()
        out_copy(0, 0, o0).start()
        gather(1, 1, g1).start()

        def step(c, b):
            gather(c, b, gsem[b]).wait()
            out_copy(c, b, osem[b]).start()
            out_copy(c - 1, b ^ 1, osem[b ^ 1]).wait()
            gather(c + 1, b ^ 1, gsem[b ^ 1]).start()

        def body(g, carry):
            step(2 * g + 1, 1)
            step(2 * g + 2, 0)
            return carry

        lax.fori_loop(0, (chunks - 2) // 2, body, 0)

        last = chunks - 1
        gather(last, 1, g1).wait()
        out_copy(last, 1, o1).start()
        out_copy(last - 1, 0, o0).wait()
        out_copy(last, 1, o1).wait()

    return k


def kernel(x, table):
    b, s = x.shape
    n = b * s
    d = table.shape[1]
    nw = 32
    idx = x.reshape(nw, n // (nw * _LANE), _LANE).astype(jnp.int32)
    out = _make_gather(n, d)(table, idx)
    return out.reshape(b, s, d)


# speedup vs baseline: 1.7838x; 1.3959x over previous
import functools

import jax
import jax.numpy as jnp
from jax import lax
from jax.experimental import pallas as pl
from jax.experimental.pallas import tpu as pltpu
from jax.experimental.pallas import tpu_sc as plsc


@functools.lru_cache(maxsize=None)
def _make_gather(n_tok, s, d):
    info = plsc.get_sparse_core_info()
    num_cores, num_subcores = info.num_cores, info.num_subcores
    nw = num_cores * num_subcores
    chunks = n_tok // nw
    assert chunks * nw == n_tok
    assert chunks % 2 == 0 and chunks >= 4
    assert s <= 128
    assert d % 128 == 0
    s_main = (s // 8) * 8
    s_tail = s - s_main

    mesh = plsc.VectorSubcoreMesh(core_axis_name="c", subcore_axis_name="s")

    out_types = [jax.ShapeDtypeStruct((n_tok, s, d), jnp.float32)]
    if s_tail:
        out_types.append(
            jax.ShapeDtypeStruct((n_tok, s_tail, d - 128), jnp.float32))

    @functools.partial(
        pl.kernel,
        mesh=mesh,
        out_type=tuple(out_types),
        scratch_types=[
            pltpu.VMEM((chunks, s), jnp.int32),
            pltpu.VMEM((2, s, d), jnp.float32),
            pltpu.VMEM((2, max(s_tail, 1), max(d - 128, 16)), jnp.float32),
            pltpu.SemaphoreType.DMA,
            pltpu.SemaphoreType.DMA,
            pltpu.SemaphoreType.DMA,
            pltpu.SemaphoreType.DMA,
        ],
    )
    def k(table_hbm, x_hbm, out_hbm, tail_hbm, idx_v, rows_v, tail_v,
          g0, g1, o0, o1):
        wid = lax.axis_index("s") * num_cores + lax.axis_index("c")
        base_tok = wid * chunks
        pltpu.sync_copy(x_hbm.at[pl.ds(base_tok, chunks)], idx_v)

        gsem = (g0, g1)
        osem = (o0, o1)

        def gathers(c, b, sem):
            parts = [pltpu.make_async_copy(
                table_hbm.at[idx_v.at[c]], rows_v.at[b], sem)]
            if s_tail:
                parts.append(pltpu.make_async_copy(
                    table_hbm.at[idx_v.at[c, pl.ds(s_main, s_tail)],
                                 pl.ds(128, d - 128)],
                    tail_v.at[b], sem))
            return parts

        def gather_start(c, b, sem):
            for p in gathers(c, b, sem):
                p.start()

        def gather_wait(c, b, sem):
            for p in gathers(c, b, sem):
                p.wait()

        def out_parts(c, b, sem):
            tok = base_tok + c
            dst = out_hbm.at[tok]
            parts = [pltpu.make_async_copy(
                rows_v.at[b, pl.ds(0, s_main)], dst.at[pl.ds(0, s_main)], sem)]
            if s_tail:
                parts.append(pltpu.make_async_copy(
                    rows_v.at[b, pl.ds(s_main, s_tail), pl.ds(0, 128)],
                    dst.at[pl.ds(s_main, s_tail), pl.ds(0, 128)], sem))
                parts.append(pltpu.make_async_copy(
                    tail_v.at[b], tail_hbm.at[tok], sem))
            return parts

        def out_start(c, b, sem):
            for p in out_parts(c, b, sem):
                p.start()

        def out_wait(c, b, sem):
            for p in out_parts(c, b, sem):
                p.wait()

        gather_start(0, 0, g0)
        gather_wait(0, 0, g0)
        out_start(0, 0, o0)
        gather_start(1, 1, g1)

        def step(c, b):
            gather_wait(c, b, gsem[b])
            out_start(c, b, osem[b])
            out_wait(c - 1, b ^ 1, osem[b ^ 1])
            gather_start(c + 1, b ^ 1, gsem[b ^ 1])

        def body(g, carry):
            step(2 * g + 1, 1)
            step(2 * g + 2, 0)
            return carry

        lax.fori_loop(0, (chunks - 2) // 2, body, 0)

        last = chunks - 1
        gather_wait(last, 1, g1)
        out_start(last, 1, o1)
        out_wait(last - 1, 0, o0)
        out_wait(last, 1, o1)

    return k


def kernel(x, table):
    n_tok, s = x.shape
    d = table.shape[1]
    s_main = (s // 8) * 8
    outs = _make_gather(n_tok, s, d)(table, x.astype(jnp.int32))
    if s_main == s:
        return outs[0]
    out, tail = outs
    return out.at[:, s_main:, 128:].set(tail)
